# baseline (speedup 1.0000x reference)
import functools

import numpy as np
import jax
import jax.numpy as jnp
from jax import lax
from jax.experimental import pallas as pl
from jax.experimental.pallas import tpu as pltpu
from jax.experimental.pallas import tpu_sc as plsc

_N = 20000
_NP = 20096
_G = 64
_TOPK = 10
_BIG = 1e30
_INIT_HI = 3.0e38
_SCALE_CLAMP = float(np.log(1000.0 / 16))
_VREGS = _NP // 16
_BLK = 8
_NBLK = _VREGS // _BLK


def _a1_body(anchors_t, deltas_t, gt, cls2, stride2, costm_ref, iouc_ref):
    ax0 = anchors_t[0:1, :]
    ay0 = anchors_t[1:2, :]
    ax1 = anchors_t[2:3, :]
    ay1 = anchors_t[3:4, :]
    d0 = deltas_t[0:1, :]
    d1 = deltas_t[1:2, :]
    d2 = deltas_t[2:3, :]
    d3 = deltas_t[3:4, :]

    widths = ax1 - ax0
    heights = ay1 - ay0
    ctr_x = ax0 + 0.5 * widths
    ctr_y = ay0 + 0.5 * heights
    dx = d0 / 10.0
    dy = d1 / 10.0
    dw = jnp.minimum(d2 / 5.0, _SCALE_CLAMP)
    dh = jnp.minimum(d3 / 5.0, _SCALE_CLAMP)
    pcx = dx * widths + ctr_x
    pcy = dy * heights + ctr_y
    pw = jnp.exp(dw) * widths
    ph = jnp.exp(dh) * heights
    px0 = pcx - 0.5 * pw
    py0 = pcy - 0.5 * ph
    px1 = pcx + 0.5 * pw
    py1 = pcy + 0.5 * ph

    x_shifts = (ax0 + ax1) / 2.0
    y_shifts = (ay0 + ay1) / 2.0

    g0 = gt[:, 0:1]
    g1 = gt[:, 1:2]
    g2 = gt[:, 2:3]
    g3 = gt[:, 3:4]

    cdist = 1.5 * stride2[...]
    gt_cx = (g0 + g2) / 2.0
    gt_cy = (g1 + g3) / 2.0
    in_cx = jnp.abs(x_shifts - gt_cx) < cdist
    in_cy = jnp.abs(y_shifts - gt_cy) < cdist
    geom = in_cx & in_cy

    area_a = (g2 - g0) * (g3 - g1)
    area_p = (px1 - px0) * (py1 - py0)
    ltx = jnp.maximum(g0, px0)
    lty = jnp.maximum(g1, py0)
    rbx = jnp.minimum(g2, px1)
    rby = jnp.minimum(g3, py1)
    whx = jnp.clip(rbx - ltx, 0.0, None)
    why = jnp.clip(rby - lty, 0.0, None)
    inter = whx * why
    union = area_a + area_p - inter
    iou = inter / jnp.maximum(union, 1e-8)

    iou_loss = -jnp.log(iou + 1e-8)
    p = jax.nn.sigmoid(cls2[...])
    cls_loss = -jnp.log(p + 1e-12)
    cost = cls_loss + 3.0 * iou_loss

    costm_ref[...] = jnp.where(geom, cost, _BIG)
    iouc_ref[...] = jnp.where(geom, iou, 0.0)


def _a2_body(anchors_t, gt, maxiou_ref):
    ax0 = anchors_t[0:1, :]
    ay0 = anchors_t[1:2, :]
    ax1 = anchors_t[2:3, :]
    ay1 = anchors_t[3:4, :]
    g0 = gt[:, 0:1]
    g1 = gt[:, 1:2]
    g2 = gt[:, 2:3]
    g3 = gt[:, 3:4]
    area_a = (g2 - g0) * (g3 - g1)
    area_b = (ax1 - ax0) * (ay1 - ay0)
    ltx = jnp.maximum(g0, ax0)
    lty = jnp.maximum(g1, ay0)
    rbx = jnp.minimum(g2, ax1)
    rby = jnp.minimum(g3, ay1)
    whx = jnp.clip(rbx - ltx, 0.0, None)
    why = jnp.clip(rby - lty, 0.0, None)
    inter = whx * why
    union = area_a + area_b - inter
    iou2 = inter / jnp.maximum(union, 1e-8)
    maxiou_ref[...] = jnp.max(iou2, axis=0, keepdims=True)


def _merge_lo(cc, v):
    vs = lax.sort(v)
    lo = jnp.minimum(cc, lax.rev(vs, (0,)))
    ns = lax.sort(lo)
    return ns, jnp.full((16,), jnp.max(ns), jnp.float32)


def _merge_hi(cc, v):
    vs = lax.sort(v)
    hi = jnp.maximum(cc, lax.rev(vs, (0,)))
    ns = lax.sort(hi)
    return ns, jnp.full((16,), jnp.min(ns), jnp.float32)


def _sc_topk_body(costm_hbm, iouc_hbm, trow_hbm,
                  cbuf0, ibuf0, cbuf1, ibuf1, orow,
                  sem0, sem1, sem2, sem3):
    wid = lax.axis_index("s") * 2 + lax.axis_index("c")
    gbase = wid * 2
    lanes = lax.iota(jnp.int32, 16)

    cp0 = pltpu.async_copy(costm_hbm.at[pl.ds(gbase * _NP, _NP)], cbuf0, sem0)
    cp1 = pltpu.async_copy(iouc_hbm.at[pl.ds(gbase * _NP, _NP)], ibuf0, sem1)
    cp2 = pltpu.async_copy(costm_hbm.at[pl.ds((gbase + 1) * _NP, _NP)], cbuf1, sem2)
    cp3 = pltpu.async_copy(iouc_hbm.at[pl.ds((gbase + 1) * _NP, _NP)], ibuf1, sem3)

    for rr, (cb, ib, cpc, cpi) in enumerate(
            ((cbuf0, ibuf0, cp0, cp1), (cbuf1, ibuf1, cp2, cp3))):
        g = gbase + rr
        cpc.wait()
        cpi.wait()

        def _scan_lo(vs, cc, w):
            for u in range(len(vs)):
                p = jnp.any(vs[u] < w)
                cc, w = lax.cond(p, lambda a, v=vs[u]: _merge_lo(a[0], v),
                                 lambda a: a, (cc, w))
            return cc, w

        def _scan_hi(vs, cc, w):
            for u in range(len(vs)):
                p = jnp.any(vs[u] > w)
                cc, w = lax.cond(p, lambda a, v=vs[u]: _merge_hi(a[0], v),
                                 lambda a: a, (cc, w))
            return cc, w

        def _sub_lo(vs, cc, w):
            mn = vs[0]
            for u in range(1, len(vs)):
                mn = jnp.minimum(mn, vs[u])
            return lax.cond(jnp.any(mn < w),
                            lambda a, vs=vs: _scan_lo(vs, a[0], a[1]),
                            lambda a: a, (cc, w))

        def _sub_hi(vs, cc, w):
            mx = vs[0]
            for u in range(1, len(vs)):
                mx = jnp.maximum(mx, vs[u])
            return lax.cond(jnp.any(mx > w),
                            lambda a, vs=vs: _scan_hi(vs, a[0], a[1]),
                            lambda a: a, (cc, w))

        _SB = 32
        _NSB = _VREGS // _SB

        def sblk(b, carry, cb=cb, ib=ib):
            curc, wc, curi, wi = carry
            base = b * (_SB * 16)
            vc = [cb[pl.ds(base + u * 16, 16)] for u in range(_SB)]
            vi = [ib[pl.ds(base + u * 16, 16)] for u in range(_SB)]
            mn = vc[0]
            mx = vi[0]
            for u in range(1, _SB):
                mn = jnp.minimum(mn, vc[u])
                mx = jnp.maximum(mx, vi[u])

            def cpath(args, vc=vc):
                cc, w = args
                for sb in range(_SB // 8):
                    cc, w = _sub_lo(vc[sb * 8:(sb + 1) * 8], cc, w)
                return cc, w

            def ipath(args, vi=vi):
                cc, w = args
                for sb in range(_SB // 8):
                    cc, w = _sub_hi(vi[sb * 8:(sb + 1) * 8], cc, w)
                return cc, w

            curc, wc = lax.cond(jnp.any(mn < wc), cpath,
                                lambda a: a, (curc, wc))
            curi, wi = lax.cond(jnp.any(mx > wi), ipath,
                                lambda a: a, (curi, wi))
            return curc, wc, curi, wi

        init = (jnp.full((16,), _INIT_HI, jnp.float32),
                jnp.full((16,), _INIT_HI, jnp.float32),
                jnp.full((16,), -1.0, jnp.float32),
                jnp.full((16,), -1.0, jnp.float32))
        curc, wc, curi, wi = lax.fori_loop(0, _NSB, sblk, init)
        tail_c = [cb[pl.ds((_NSB * _SB + u) * 16, 16)]
                  for u in range(_VREGS - _NSB * _SB)]
        tail_i = [ib[pl.ds((_NSB * _SB + u) * 16, 16)]
                  for u in range(_VREGS - _NSB * _SB)]
        curc, wc = _sub_lo(tail_c, curc, wc)
        curi, wi = _sub_hi(tail_i, curi, wi)

        desc = lax.rev(curi, (0,))
        s = jnp.float32(0.0)
        for i in range(_TOPK):
            s = s + jnp.sum(jnp.where(lanes == i, desc, 0.0))
        ssp = jnp.maximum(s, 1.0)
        ki = plsc.all_reduce_population_count(
            ssp >= (lanes + 1).astype(jnp.float32))

        t = jnp.sum(jnp.where(lanes == ki - 1, curc, 0.0))
        cnt_lt = jnp.sum(jnp.where(curc < t, 1, 0))
        r = ki - cnt_lt
        nextv = jnp.sum(jnp.where(lanes == ki, curc, 0.0))
        straddle = (nextv == t) & (t < 1e29)

        def tie(_, cb=cb, t_=None):
            def step(j, carry):
                cnt, found = carry
                v = cb[pl.ds(j * 16, 16)]
                m = v == t
                mi = jnp.where(m, 1, 0)
                pref = jnp.cumsum(mi)
                hit = m & ((cnt + pref) == r)
                idxv = jnp.where(hit, lanes + j * 16, _NP)
                found = jnp.minimum(found, jnp.min(idxv))
                return cnt + jnp.sum(mi), found
            _, found = lax.fori_loop(0, _VREGS, step,
                                     (jnp.int32(0), jnp.int32(_NP)))
            return found

        idx_cut = lax.cond(straddle, tie, lambda _: jnp.int32(_NP), None)
        idx_cut = jnp.where(t < 1e29, idx_cut, -1)

        dbg = jnp.where(lanes == 2, s,
                        jnp.where(lanes == 3, ki.astype(jnp.float32),
                                  jnp.where(lanes >= 8, curc, 0.0)))
        orow[...] = jnp.where(lanes == 0, t,
                              jnp.where(lanes == 1,
                                        idx_cut.astype(jnp.float32), dbg))
        pltpu.sync_copy(orow, trow_hbm.at[pl.ds(g * 16, 16)])


@functools.lru_cache(maxsize=1)
def _make_sc_topk():
  return pl.kernel(
    _sc_topk_body,
    out_type=jax.ShapeDtypeStruct((_G * 16,), jnp.float32),
    mesh=plsc.VectorSubcoreMesh(core_axis_name="c", subcore_axis_name="s"),
    scratch_types=[
        pltpu.VMEM((_NP,), jnp.float32),
        pltpu.VMEM((_NP,), jnp.float32),
        pltpu.VMEM((_NP,), jnp.float32),
        pltpu.VMEM((_NP,), jnp.float32),
        pltpu.VMEM((16,), jnp.float32),
        pltpu.SemaphoreType.DMA,
        pltpu.SemaphoreType.DMA,
        pltpu.SemaphoreType.DMA,
        pltpu.SemaphoreType.DMA,
    ],
    compiler_params=pltpu.CompilerParams(needs_layout_passes=False),
  )


def _c_body(costm_ref, iouc_ref, trow_ref, maxiou_ref, gt_ref, out_ref):
    cm = costm_ref[...]
    t = trow_ref[:, 0:1]
    idx_cut = trow_ref[:, 1:2].astype(jnp.int32)
    col_ids = jax.lax.broadcasted_iota(jnp.int32, (_G, _NP), 1)
    row_ids = jax.lax.broadcasted_iota(jnp.int32, (_G, _NP), 0)

    matched = (cm < t) | ((cm == t) & (col_ids <= idx_cut))

    ones = jnp.where(matched, 1.0, 0.0)
    nmatch = jnp.sum(ones, axis=0, keepdims=True)
    multi = nmatch > 1.0
    cmin = jnp.min(cm, axis=0, keepdims=True)
    best_gt = jnp.min(jnp.where(cm == cmin, row_ids, _G), axis=0, keepdims=True)
    onehot = row_ids == best_gt
    matching = matched & ~(multi & ~onehot)
    matchf = jnp.where(matching, 1.0, 0.0)

    fg = jnp.sum(matchf, axis=0, keepdims=True) > 0.0
    pred_ious = jnp.sum(jnp.where(matching, iouc_ref[...], 0.0),
                        axis=0, keepdims=True)

    label = jnp.where(fg, 1.0,
                      jnp.where(maxiou_ref[...] >= 0.3, -1.0, 0.0))

    g0 = gt_ref[:, 0:1]
    g1 = gt_ref[:, 1:2]
    g2 = gt_ref[:, 2:3]
    g3 = gt_ref[:, 3:4]
    bx0 = jnp.sum(matchf * g0, axis=0, keepdims=True)
    bx1 = jnp.sum(matchf * g1, axis=0, keepdims=True)
    bx2 = jnp.sum(matchf * g2, axis=0, keepdims=True)
    bx3 = jnp.sum(matchf * g3, axis=0, keepdims=True)

    zero = jnp.zeros((1, _NP), jnp.float32)
    out_ref[:, :] = jnp.concatenate(
        [label, pred_ious, bx0, bx1, bx2, bx3, zero, zero], axis=0)


@jax.jit
def kernel(anchors, pred_deltas, gt_boxes, cls_preds, expanded_strides, gt_classes):
    del gt_classes
    pad = ((0, 0), (0, _NP - _N))
    anchors_t = jnp.pad(anchors.T, pad)
    deltas_t = jnp.pad(pred_deltas.T, pad)
    cls2 = jnp.pad(cls_preds.reshape(1, _N), pad)
    stride2 = jnp.pad(expanded_strides.reshape(1, _N), pad)

    costm, iouc = pl.pallas_call(
        _a1_body,
        out_shape=[jax.ShapeDtypeStruct((_G, _NP), jnp.float32),
                   jax.ShapeDtypeStruct((_G, _NP), jnp.float32)],
    )(anchors_t, deltas_t, gt_boxes, cls2, stride2)

    maxiou = pl.pallas_call(
        _a2_body,
        out_shape=jax.ShapeDtypeStruct((1, _NP), jnp.float32),
    )(anchors_t, gt_boxes)

    trow_f = _make_sc_topk()(jnp.reshape(costm, (_G * _NP,)),
                             jnp.reshape(iouc, (_G * _NP,)))
    trow = jnp.reshape(trow_f, (_G, 16))

    out = pl.pallas_call(
        _c_body,
        out_shape=jax.ShapeDtypeStruct((8, _NP), jnp.float32),
    )(costm, iouc, trow, maxiou, gt_boxes)
    return out[:6, :_N].T


# speedup vs baseline: 1.9620x; 1.9620x over previous
import functools

import numpy as np
import jax
import jax.numpy as jnp
from jax import lax
from jax.experimental import pallas as pl
from jax.experimental.pallas import tpu as pltpu
from jax.experimental.pallas import tpu_sc as plsc

_N = 20000
_CH = 1280
_NF = 16
_NP = _CH * _NF
_G = 64
_TOPK = 10
_BIG = 1e30
_INIT_HI = 3.0e38
_SCALE_CLAMP = float(np.log(1000.0 / 16))
_VREGS = _NP // 16
_DVREGS = _CH // 16


def _a1_body(anchors_t, deltas_t, gt, cls2, stride2,
             costm_ref, iouc_ref, cdig_ref, idig_ref):
    ax0 = anchors_t[0:1, :]
    ay0 = anchors_t[1:2, :]
    ax1 = anchors_t[2:3, :]
    ay1 = anchors_t[3:4, :]
    d0 = deltas_t[0:1, :]
    d1 = deltas_t[1:2, :]
    d2 = deltas_t[2:3, :]
    d3 = deltas_t[3:4, :]

    widths = ax1 - ax0
    heights = ay1 - ay0
    ctr_x = ax0 + 0.5 * widths
    ctr_y = ay0 + 0.5 * heights
    dx = d0 / 10.0
    dy = d1 / 10.0
    dw = jnp.minimum(d2 / 5.0, _SCALE_CLAMP)
    dh = jnp.minimum(d3 / 5.0, _SCALE_CLAMP)
    pcx = dx * widths + ctr_x
    pcy = dy * heights + ctr_y
    pw = jnp.exp(dw) * widths
    ph = jnp.exp(dh) * heights
    px0 = pcx - 0.5 * pw
    py0 = pcy - 0.5 * ph
    px1 = pcx + 0.5 * pw
    py1 = pcy + 0.5 * ph

    x_shifts = (ax0 + ax1) / 2.0
    y_shifts = (ay0 + ay1) / 2.0

    g0 = gt[:, 0:1]
    g1 = gt[:, 1:2]
    g2 = gt[:, 2:3]
    g3 = gt[:, 3:4]

    cdist = 1.5 * stride2[...]
    gt_cx = (g0 + g2) / 2.0
    gt_cy = (g1 + g3) / 2.0
    in_cx = jnp.abs(x_shifts - gt_cx) < cdist
    in_cy = jnp.abs(y_shifts - gt_cy) < cdist
    geom = in_cx & in_cy

    area_a = (g2 - g0) * (g3 - g1)
    area_p = (px1 - px0) * (py1 - py0)
    ltx = jnp.maximum(g0, px0)
    lty = jnp.maximum(g1, py0)
    rbx = jnp.minimum(g2, px1)
    rby = jnp.minimum(g3, py1)
    whx = jnp.clip(rbx - ltx, 0.0, None)
    why = jnp.clip(rby - lty, 0.0, None)
    inter = whx * why
    union = area_a + area_p - inter
    iou = inter / jnp.maximum(union, 1e-8)

    iou_loss = -jnp.log(iou + 1e-8)
    p = jax.nn.sigmoid(cls2[...])
    cls_loss = -jnp.log(p + 1e-12)
    cost = cls_loss + 3.0 * iou_loss

    cm = jnp.where(geom, cost, _BIG)
    ic = jnp.where(geom, iou, 0.0)
    costm_ref[...] = cm
    iouc_ref[...] = ic

    dc = cm[:, 0:_CH]
    di = ic[:, 0:_CH]
    for j in range(1, _NF):
        dc = jnp.minimum(dc, cm[:, j * _CH:(j + 1) * _CH])
        di = jnp.maximum(di, ic[:, j * _CH:(j + 1) * _CH])
    cdig_ref[...] = dc
    idig_ref[...] = di


def _a2_body(anchors_t, gt, maxiou_ref):
    ax0 = anchors_t[0:1, :]
    ay0 = anchors_t[1:2, :]
    ax1 = anchors_t[2:3, :]
    ay1 = anchors_t[3:4, :]
    g0 = gt[:, 0:1]
    g1 = gt[:, 1:2]
    g2 = gt[:, 2:3]
    g3 = gt[:, 3:4]
    area_a = (g2 - g0) * (g3 - g1)
    area_b = (ax1 - ax0) * (ay1 - ay0)
    ltx = jnp.maximum(g0, ax0)
    lty = jnp.maximum(g1, ay0)
    rbx = jnp.minimum(g2, ax1)
    rby = jnp.minimum(g3, ay1)
    whx = jnp.clip(rbx - ltx, 0.0, None)
    why = jnp.clip(rby - lty, 0.0, None)
    inter = whx * why
    union = area_a + area_b - inter
    iou2 = inter / jnp.maximum(union, 1e-8)
    maxiou_ref[...] = jnp.max(iou2, axis=0, keepdims=True)


def _merge_lo(cc, v):
    vs = lax.sort(v)
    lo = jnp.minimum(cc, lax.rev(vs, (0,)))
    ns = lax.sort(lo)
    return ns, jnp.full((16,), jnp.max(ns), jnp.float32)


def _merge_hi(cc, v):
    vs = lax.sort(v)
    hi = jnp.maximum(cc, lax.rev(vs, (0,)))
    ns = lax.sort(hi)
    return ns, jnp.full((16,), jnp.min(ns), jnp.float32)


def _merge_lo_kv(cc, pc, v, pv):
    ks, vs = plsc.sort_key_val(v, pv)
    rk = lax.rev(ks, (0,))
    rv = lax.rev(vs, (0,))
    m = cc <= rk
    lo = jnp.where(m, cc, rk)
    lp = jnp.where(m, pc, rv)
    ks2, vs2 = plsc.sort_key_val(lo, lp)
    return ks2, vs2, jnp.full((16,), jnp.max(ks2), jnp.float32)


def _merge_hi_kv(cc, pc, v, pv):
    ks, vs = plsc.sort_key_val(v, pv)
    rk = lax.rev(ks, (0,))
    rv = lax.rev(vs, (0,))
    m = cc >= rk
    hi = jnp.where(m, cc, rk)
    hp = jnp.where(m, pc, rv)
    ks2, vs2 = plsc.sort_key_val(hi, hp)
    return ks2, vs2, jnp.full((16,), jnp.min(ks2), jnp.float32)


def _sc_topk_body(costm_hbm, iouc_hbm, cdig_hbm, idig_hbm, trow_hbm,
                  cbuf0, ibuf0, cbuf1, ibuf1,
                  cdg0, idg0, cdg1, idg1, orow,
                  sem0, sem1, sem2, sem3, sem4, sem5, sem6, sem7):
    wid = lax.axis_index("s") * 2 + lax.axis_index("c")
    gbase = wid * 2
    lanes = lax.iota(jnp.int32, 16)

    cps = []
    for rr, (cb, ib, cd, idg, s_a, s_b, s_c, s_d) in enumerate((
            (cbuf0, ibuf0, cdg0, idg0, sem0, sem1, sem2, sem3),
            (cbuf1, ibuf1, cdg1, idg1, sem4, sem5, sem6, sem7))):
        g = gbase + rr
        cps.append((
            pltpu.async_copy(cdig_hbm.at[pl.ds(g * _CH, _CH)], cd, s_a),
            pltpu.async_copy(idig_hbm.at[pl.ds(g * _CH, _CH)], idg, s_b),
            pltpu.async_copy(costm_hbm.at[pl.ds(g * _NP, _NP)], cb, s_c),
            pltpu.async_copy(iouc_hbm.at[pl.ds(g * _NP, _NP)], ib, s_d),
        ))

    for rr, (cb, ib, cd, idg) in enumerate(
            ((cbuf0, ibuf0, cdg0, idg0), (cbuf1, ibuf1, cdg1, idg1))):
        g = gbase + rr
        for cp in cps[rr]:
            cp.wait()

        def dscan(p, carry, cd=cd, idg=idg):
            curc, wc, posc, curi, wi, posi = carry
            pv = lanes + p * 16
            vc = cd[pl.ds(p * 16, 16)]
            vi = idg[pl.ds(p * 16, 16)]
            curc, posc, wc = lax.cond(
                jnp.any(vc < wc),
                lambda a, v=vc, q=pv: _merge_lo_kv(a[0], a[1], v, q),
                lambda a: (a[0], a[1], a[2]), (curc, posc, wc))
            curi, posi, wi = lax.cond(
                jnp.any(vi > wi),
                lambda a, v=vi, q=pv: _merge_hi_kv(a[0], a[1], v, q),
                lambda a: (a[0], a[1], a[2]), (curi, posi, wi))
            return curc, wc, posc, curi, wi, posi

        init = (jnp.full((16,), _INIT_HI, jnp.float32),
                jnp.full((16,), _INIT_HI, jnp.float32),
                jnp.full((16,), 0, jnp.int32),
                jnp.full((16,), -1.0, jnp.float32),
                jnp.full((16,), -1.0, jnp.float32),
                jnp.full((16,), 0, jnp.int32))
        curc, wc, posc, curi, wi, posi = lax.fori_loop(0, _DVREGS, dscan, init)

        curc = jnp.full((16,), _INIT_HI, jnp.float32)
        wc = jnp.full((16,), _INIT_HI, jnp.float32)
        curi = jnp.full((16,), -1.0, jnp.float32)
        wi = jnp.full((16,), -1.0, jnp.float32)
        for j in range(_NF):
            gc = plsc.load_gather(cb, [posc + j * _CH])
            curc, wc = lax.cond(jnp.any(gc < wc),
                                lambda a, v=gc: _merge_lo(a[0], v),
                                lambda a: a, (curc, wc))
            gi = plsc.load_gather(ib, [posi + j * _CH])
            curi, wi = lax.cond(jnp.any(gi > wi),
                                lambda a, v=gi: _merge_hi(a[0], v),
                                lambda a: a, (curi, wi))

        desc = lax.rev(curi, (0,))
        s = jnp.float32(0.0)
        for i in range(_TOPK):
            s = s + jnp.sum(jnp.where(lanes == i, desc, 0.0))
        ssp = jnp.maximum(s, 1.0)
        ki = plsc.all_reduce_population_count(
            ssp >= (lanes + 1).astype(jnp.float32))

        t = jnp.sum(jnp.where(lanes == ki - 1, curc, 0.0))
        cnt_lt = jnp.sum(jnp.where(curc < t, 1, 0))
        r = ki - cnt_lt
        nextv = jnp.sum(jnp.where(lanes == ki, curc, 0.0))
        straddle = (nextv == t) & (t < 1e29)

        def tie(_, cb=cb):
            def step(p, carry):
                cnt, found = carry
                v = cb[pl.ds(p * 16, 16)]
                m = v == t
                mi = jnp.where(m, 1, 0)
                pref = jnp.cumsum(mi)
                hit = m & ((cnt + pref) == r)
                idxv = jnp.where(hit, lanes + p * 16, _NP)
                found = jnp.minimum(found, jnp.min(idxv))
                return cnt + jnp.sum(mi), found
            _, found = lax.fori_loop(0, _VREGS, step,
                                     (jnp.int32(0), jnp.int32(_NP)))
            return found

        idx_cut = lax.cond(straddle, tie, lambda _: jnp.int32(_NP), None)
        idx_cut = jnp.where(t < 1e29, idx_cut, -1)

        dbg = jnp.where(lanes == 2, s,
                        jnp.where(lanes == 3, ki.astype(jnp.float32),
                                  jnp.where(lanes >= 8, curc, desc)))
        orow[...] = jnp.where(lanes == 0, t,
                              jnp.where(lanes == 1,
                                        idx_cut.astype(jnp.float32), dbg))
        pltpu.sync_copy(orow, trow_hbm.at[pl.ds(g * 16, 16)])


@functools.lru_cache(maxsize=1)
def _make_sc_topk():
  return pl.kernel(
    _sc_topk_body,
    out_type=jax.ShapeDtypeStruct((_G * 16,), jnp.float32),
    mesh=plsc.VectorSubcoreMesh(core_axis_name="c", subcore_axis_name="s"),
    scratch_types=[
        pltpu.VMEM((_NP,), jnp.float32),
        pltpu.VMEM((_NP,), jnp.float32),
        pltpu.VMEM((_NP,), jnp.float32),
        pltpu.VMEM((_NP,), jnp.float32),
        pltpu.VMEM((_CH,), jnp.float32),
        pltpu.VMEM((_CH,), jnp.float32),
        pltpu.VMEM((_CH,), jnp.float32),
        pltpu.VMEM((_CH,), jnp.float32),
        pltpu.VMEM((16,), jnp.float32),
        pltpu.SemaphoreType.DMA,
        pltpu.SemaphoreType.DMA,
        pltpu.SemaphoreType.DMA,
        pltpu.SemaphoreType.DMA,
        pltpu.SemaphoreType.DMA,
        pltpu.SemaphoreType.DMA,
        pltpu.SemaphoreType.DMA,
        pltpu.SemaphoreType.DMA,
    ],
    compiler_params=pltpu.CompilerParams(needs_layout_passes=False),
  )


def _c_body(costm_ref, iouc_ref, trow_ref, maxiou_ref, gt_ref, out_ref):
    cm = costm_ref[...]
    t = trow_ref[:, 0:1]
    idx_cut = trow_ref[:, 1:2].astype(jnp.int32)
    col_ids = jax.lax.broadcasted_iota(jnp.int32, (_G, _NP), 1)
    row_ids = jax.lax.broadcasted_iota(jnp.int32, (_G, _NP), 0)

    matched = (cm < t) | ((cm == t) & (col_ids <= idx_cut))

    ones = jnp.where(matched, 1.0, 0.0)
    nmatch = jnp.sum(ones, axis=0, keepdims=True)
    multi = nmatch > 1.0
    cmin = jnp.min(cm, axis=0, keepdims=True)
    best_gt = jnp.min(jnp.where(cm == cmin, row_ids, _G), axis=0, keepdims=True)
    onehot = row_ids == best_gt
    matching = matched & ~(multi & ~onehot)
    matchf = jnp.where(matching, 1.0, 0.0)

    fg = jnp.sum(matchf, axis=0, keepdims=True) > 0.0
    pred_ious = jnp.sum(jnp.where(matching, iouc_ref[...], 0.0),
                        axis=0, keepdims=True)

    label = jnp.where(fg, 1.0,
                      jnp.where(maxiou_ref[...] >= 0.3, -1.0, 0.0))

    g0 = gt_ref[:, 0:1]
    g1 = gt_ref[:, 1:2]
    g2 = gt_ref[:, 2:3]
    g3 = gt_ref[:, 3:4]
    bx0 = jnp.sum(matchf * g0, axis=0, keepdims=True)
    bx1 = jnp.sum(matchf * g1, axis=0, keepdims=True)
    bx2 = jnp.sum(matchf * g2, axis=0, keepdims=True)
    bx3 = jnp.sum(matchf * g3, axis=0, keepdims=True)

    zero = jnp.zeros((1, _NP), jnp.float32)
    out_ref[:, :] = jnp.concatenate(
        [label, pred_ious, bx0, bx1, bx2, bx3, zero, zero], axis=0)


@jax.jit
def kernel(anchors, pred_deltas, gt_boxes, cls_preds, expanded_strides, gt_classes):
    del gt_classes
    pad = ((0, 0), (0, _NP - _N))
    anchors_t = jnp.pad(anchors.T, pad)
    deltas_t = jnp.pad(pred_deltas.T, pad)
    cls2 = jnp.pad(cls_preds.reshape(1, _N), pad)
    stride2 = jnp.pad(expanded_strides.reshape(1, _N), pad)

    costm, iouc, cdig, idig = pl.pallas_call(
        _a1_body,
        out_shape=[jax.ShapeDtypeStruct((_G, _NP), jnp.float32),
                   jax.ShapeDtypeStruct((_G, _NP), jnp.float32),
                   jax.ShapeDtypeStruct((_G, _CH), jnp.float32),
                   jax.ShapeDtypeStruct((_G, _CH), jnp.float32)],
    )(anchors_t, deltas_t, gt_boxes, cls2, stride2)

    maxiou = pl.pallas_call(
        _a2_body,
        out_shape=jax.ShapeDtypeStruct((1, _NP), jnp.float32),
    )(anchors_t, gt_boxes)

    trow_f = _make_sc_topk()(jnp.reshape(costm, (_G * _NP,)),
                             jnp.reshape(iouc, (_G * _NP,)),
                             jnp.reshape(cdig, (_G * _CH,)),
                             jnp.reshape(idig, (_G * _CH,)))
    trow = jnp.reshape(trow_f, (_G, 16))

    out = pl.pallas_call(
        _c_body,
        out_shape=jax.ShapeDtypeStruct((8, _NP), jnp.float32),
    )(costm, iouc, trow, maxiou, gt_boxes)
    return out[:6, :_N].T


# speedup vs baseline: 2.2054x; 1.1241x over previous
import functools

import numpy as np
import jax
import jax.numpy as jnp
from jax import lax
from jax.experimental import pallas as pl
from jax.experimental.pallas import tpu as pltpu
from jax.experimental.pallas import tpu_sc as plsc

_N = 20000
_CH = 1280
_NF = 16
_NP = _CH * _NF
_G = 64
_TOPK = 10
_BIG = 1e30
_INIT_HI = 3.0e38
_SCALE_CLAMP = float(np.log(1000.0 / 16))
_VREGS = _NP // 16
_DVREGS = _CH // 16


def _a1_body(anchors_t, deltas_t, gt, cls2, stride2,
             costm_ref, iouc_ref, cdig_ref, idig_ref, maxiou_ref):
    ax0 = anchors_t[0:1, :]
    ay0 = anchors_t[1:2, :]
    ax1 = anchors_t[2:3, :]
    ay1 = anchors_t[3:4, :]
    d0 = deltas_t[0:1, :]
    d1 = deltas_t[1:2, :]
    d2 = deltas_t[2:3, :]
    d3 = deltas_t[3:4, :]

    widths = ax1 - ax0
    heights = ay1 - ay0
    ctr_x = ax0 + 0.5 * widths
    ctr_y = ay0 + 0.5 * heights
    dx = d0 / 10.0
    dy = d1 / 10.0
    dw = jnp.minimum(d2 / 5.0, _SCALE_CLAMP)
    dh = jnp.minimum(d3 / 5.0, _SCALE_CLAMP)
    pcx = dx * widths + ctr_x
    pcy = dy * heights + ctr_y
    pw = jnp.exp(dw) * widths
    ph = jnp.exp(dh) * heights
    px0 = pcx - 0.5 * pw
    py0 = pcy - 0.5 * ph
    px1 = pcx + 0.5 * pw
    py1 = pcy + 0.5 * ph

    x_shifts = (ax0 + ax1) / 2.0
    y_shifts = (ay0 + ay1) / 2.0

    g0 = gt[:, 0:1]
    g1 = gt[:, 1:2]
    g2 = gt[:, 2:3]
    g3 = gt[:, 3:4]

    cdist = 1.5 * stride2[...]
    gt_cx = (g0 + g2) / 2.0
    gt_cy = (g1 + g3) / 2.0
    in_cx = jnp.abs(x_shifts - gt_cx) < cdist
    in_cy = jnp.abs(y_shifts - gt_cy) < cdist
    geom = in_cx & in_cy

    area_a = (g2 - g0) * (g3 - g1)
    area_p = (px1 - px0) * (py1 - py0)
    ltx = jnp.maximum(g0, px0)
    lty = jnp.maximum(g1, py0)
    rbx = jnp.minimum(g2, px1)
    rby = jnp.minimum(g3, py1)
    whx = jnp.clip(rbx - ltx, 0.0, None)
    why = jnp.clip(rby - lty, 0.0, None)
    inter = whx * why
    union = area_a + area_p - inter
    iou = inter / jnp.maximum(union, 1e-8)

    iou_loss = -jnp.log(iou + 1e-8)
    p = jax.nn.sigmoid(cls2[...])
    cls_loss = -jnp.log(p + 1e-12)
    cost = cls_loss + 3.0 * iou_loss

    cm = jnp.where(geom, cost, _BIG)
    ic = jnp.where(geom, iou, 0.0)
    costm_ref[...] = cm
    iouc_ref[...] = ic

    dc = cm[:, 0:_CH]
    di = ic[:, 0:_CH]
    for j in range(1, _NF):
        dc = jnp.minimum(dc, cm[:, j * _CH:(j + 1) * _CH])
        di = jnp.maximum(di, ic[:, j * _CH:(j + 1) * _CH])
    cdig_ref[...] = dc
    idig_ref[...] = di

    area_b = (ax1 - ax0) * (ay1 - ay0)
    ltx2 = jnp.maximum(g0, ax0)
    lty2 = jnp.maximum(g1, ay0)
    rbx2 = jnp.minimum(g2, ax1)
    rby2 = jnp.minimum(g3, ay1)
    whx2 = jnp.clip(rbx2 - ltx2, 0.0, None)
    why2 = jnp.clip(rby2 - lty2, 0.0, None)
    inter2 = whx2 * why2
    union2 = area_a + area_b - inter2
    iou2 = inter2 / jnp.maximum(union2, 1e-8)
    maxiou_ref[...] = jnp.max(iou2, axis=0, keepdims=True)


def _merge_lo(cc, v):
    vs = lax.sort(v)
    lo = jnp.minimum(cc, lax.rev(vs, (0,)))
    ns = lax.sort(lo)
    return ns, jnp.full((16,), jnp.max(ns), jnp.float32)


def _merge_hi(cc, v):
    vs = lax.sort(v)
    hi = jnp.maximum(cc, lax.rev(vs, (0,)))
    ns = lax.sort(hi)
    return ns, jnp.full((16,), jnp.min(ns), jnp.float32)


def _merge_lo_kv(cc, pc, v, pv):
    ks, vs = plsc.sort_key_val(v, pv)
    rk = lax.rev(ks, (0,))
    rv = lax.rev(vs, (0,))
    m = cc <= rk
    lo = jnp.where(m, cc, rk)
    lp = jnp.where(m, pc, rv)
    ks2, vs2 = plsc.sort_key_val(lo, lp)
    return ks2, vs2, jnp.full((16,), jnp.max(ks2), jnp.float32)


def _merge_hi_kv(cc, pc, v, pv):
    ks, vs = plsc.sort_key_val(v, pv)
    rk = lax.rev(ks, (0,))
    rv = lax.rev(vs, (0,))
    m = cc >= rk
    hi = jnp.where(m, cc, rk)
    hp = jnp.where(m, pc, rv)
    ks2, vs2 = plsc.sort_key_val(hi, hp)
    return ks2, vs2, jnp.full((16,), jnp.min(ks2), jnp.float32)


def _sc_topk_body(costm_hbm, iouc_hbm, cdig_hbm, idig_hbm, trow_hbm,
                  cbuf0, ibuf0, cbuf1, ibuf1,
                  cdg0, idg0, cdg1, idg1, orow,
                  sem0, sem1, sem2, sem3, sem4, sem5, sem6, sem7):
    wid = lax.axis_index("s") * 2 + lax.axis_index("c")
    gbase = wid * 2
    lanes = lax.iota(jnp.int32, 16)

    cps = []
    for rr, (cb, ib, cd, idg, s_a, s_b, s_c, s_d) in enumerate((
            (cbuf0, ibuf0, cdg0, idg0, sem0, sem1, sem2, sem3),
            (cbuf1, ibuf1, cdg1, idg1, sem4, sem5, sem6, sem7))):
        g = gbase + rr
        cps.append((
            pltpu.async_copy(cdig_hbm.at[g], cd, s_a),
            pltpu.async_copy(idig_hbm.at[g], idg, s_b),
            pltpu.async_copy(costm_hbm.at[g], cb, s_c),
            pltpu.async_copy(iouc_hbm.at[g], ib, s_d),
        ))

    for rr, (cb, ib, cd, idg) in enumerate(
            ((cbuf0, ibuf0, cdg0, idg0), (cbuf1, ibuf1, cdg1, idg1))):
        g = gbase + rr
        for cp in cps[rr]:
            cp.wait()

        def dscan(p, carry, cd=cd, idg=idg):
            curc, wc, posc, curi, wi, posi = carry
            pv = lanes + p * 16
            vc = cd[pl.ds(p * 16, 16)]
            vi = idg[pl.ds(p * 16, 16)]
            curc, posc, wc = lax.cond(
                jnp.any(vc < wc),
                lambda a, v=vc, q=pv: _merge_lo_kv(a[0], a[1], v, q),
                lambda a: (a[0], a[1], a[2]), (curc, posc, wc))
            curi, posi, wi = lax.cond(
                jnp.any(vi > wi),
                lambda a, v=vi, q=pv: _merge_hi_kv(a[0], a[1], v, q),
                lambda a: (a[0], a[1], a[2]), (curi, posi, wi))
            return curc, wc, posc, curi, wi, posi

        init = (jnp.full((16,), _INIT_HI, jnp.float32),
                jnp.full((16,), _INIT_HI, jnp.float32),
                jnp.full((16,), 0, jnp.int32),
                jnp.full((16,), -1.0, jnp.float32),
                jnp.full((16,), -1.0, jnp.float32),
                jnp.full((16,), 0, jnp.int32))
        curc, wc, posc, curi, wi, posi = lax.fori_loop(0, _DVREGS, dscan, init)

        curc = jnp.full((16,), _INIT_HI, jnp.float32)
        wc = jnp.full((16,), _INIT_HI, jnp.float32)
        curi = jnp.full((16,), -1.0, jnp.float32)
        wi = jnp.full((16,), -1.0, jnp.float32)
        for j in range(_NF):
            gc = plsc.load_gather(cb, [posc + j * _CH])
            curc, wc = lax.cond(jnp.any(gc < wc),
                                lambda a, v=gc: _merge_lo(a[0], v),
                                lambda a: a, (curc, wc))
            gi = plsc.load_gather(ib, [posi + j * _CH])
            curi, wi = lax.cond(jnp.any(gi > wi),
                                lambda a, v=gi: _merge_hi(a[0], v),
                                lambda a: a, (curi, wi))

        desc = lax.rev(curi, (0,))
        s = jnp.float32(0.0)
        for i in range(_TOPK):
            s = s + jnp.sum(jnp.where(lanes == i, desc, 0.0))
        ssp = jnp.maximum(s, 1.0)
        ki = plsc.all_reduce_population_count(
            ssp >= (lanes + 1).astype(jnp.float32))

        t = jnp.sum(jnp.where(lanes == ki - 1, curc, 0.0))
        cnt_lt = jnp.sum(jnp.where(curc < t, 1, 0))
        r = ki - cnt_lt
        nextv = jnp.sum(jnp.where(lanes == ki, curc, 0.0))
        straddle = (nextv == t) & (t < 1e29)

        def tie(_, cb=cb):
            def step(p, carry):
                cnt, found = carry
                v = cb[pl.ds(p * 16, 16)]
                m = v == t
                mi = jnp.where(m, 1, 0)
                pref = jnp.cumsum(mi)
                hit = m & ((cnt + pref) == r)
                idxv = jnp.where(hit, lanes + p * 16, _NP)
                found = jnp.minimum(found, jnp.min(idxv))
                return cnt + jnp.sum(mi), found
            _, found = lax.fori_loop(0, _VREGS, step,
                                     (jnp.int32(0), jnp.int32(_NP)))
            return found

        idx_cut = lax.cond(straddle, tie, lambda _: jnp.int32(_NP), None)
        idx_cut = jnp.where(t < 1e29, idx_cut, -1)

        dbg = jnp.where(lanes == 2, s,
                        jnp.where(lanes == 3, ki.astype(jnp.float32),
                                  jnp.where(lanes >= 8, curc, desc)))
        orow[...] = jnp.where(lanes == 0, t,
                              jnp.where(lanes == 1,
                                        idx_cut.astype(jnp.float32), dbg))
        pltpu.sync_copy(orow, trow_hbm.at[pl.ds(g * 16, 16)])


@functools.lru_cache(maxsize=1)
def _make_sc_topk():
  return pl.kernel(
    _sc_topk_body,
    out_type=jax.ShapeDtypeStruct((_G * 16,), jnp.float32),
    mesh=plsc.VectorSubcoreMesh(core_axis_name="c", subcore_axis_name="s"),
    scratch_types=[
        pltpu.VMEM((_NP,), jnp.float32),
        pltpu.VMEM((_NP,), jnp.float32),
        pltpu.VMEM((_NP,), jnp.float32),
        pltpu.VMEM((_NP,), jnp.float32),
        pltpu.VMEM((_CH,), jnp.float32),
        pltpu.VMEM((_CH,), jnp.float32),
        pltpu.VMEM((_CH,), jnp.float32),
        pltpu.VMEM((_CH,), jnp.float32),
        pltpu.VMEM((16,), jnp.float32),
        pltpu.SemaphoreType.DMA,
        pltpu.SemaphoreType.DMA,
        pltpu.SemaphoreType.DMA,
        pltpu.SemaphoreType.DMA,
        pltpu.SemaphoreType.DMA,
        pltpu.SemaphoreType.DMA,
        pltpu.SemaphoreType.DMA,
        pltpu.SemaphoreType.DMA,
    ],
    compiler_params=pltpu.CompilerParams(needs_layout_passes=False),
  )


def _c_body(costm_ref, iouc_ref, trow_ref, maxiou_ref, gt_ref, out_ref):
    cm = costm_ref[...]
    t = trow_ref[:, 0:1]
    idx_cut = trow_ref[:, 1:2].astype(jnp.int32)
    col_ids = jax.lax.broadcasted_iota(jnp.int32, (_G, _NP), 1)
    row_ids = jax.lax.broadcasted_iota(jnp.int32, (_G, _NP), 0)

    matched = (cm < t) | ((cm == t) & (col_ids <= idx_cut))

    ones = jnp.where(matched, 1.0, 0.0)
    nmatch = jnp.sum(ones, axis=0, keepdims=True)
    multi = nmatch > 1.0
    cmin = jnp.min(cm, axis=0, keepdims=True)
    best_gt = jnp.min(jnp.where(cm == cmin, row_ids, _G), axis=0, keepdims=True)
    onehot = row_ids == best_gt
    matching = matched & ~(multi & ~onehot)
    matchf = jnp.where(matching, 1.0, 0.0)

    fg = jnp.sum(matchf, axis=0, keepdims=True) > 0.0
    pred_ious = jnp.sum(jnp.where(matching, iouc_ref[...], 0.0),
                        axis=0, keepdims=True)

    label = jnp.where(fg, 1.0,
                      jnp.where(maxiou_ref[...] >= 0.3, -1.0, 0.0))

    g0 = gt_ref[:, 0:1]
    g1 = gt_ref[:, 1:2]
    g2 = gt_ref[:, 2:3]
    g3 = gt_ref[:, 3:4]
    bx0 = jnp.sum(matchf * g0, axis=0, keepdims=True)
    bx1 = jnp.sum(matchf * g1, axis=0, keepdims=True)
    bx2 = jnp.sum(matchf * g2, axis=0, keepdims=True)
    bx3 = jnp.sum(matchf * g3, axis=0, keepdims=True)

    zero = jnp.zeros((1, _NP), jnp.float32)
    out_ref[:, :] = jnp.concatenate(
        [label, pred_ious, bx0, bx1, bx2, bx3, zero, zero], axis=0)


@jax.jit
def kernel(anchors, pred_deltas, gt_boxes, cls_preds, expanded_strides, gt_classes):
    del gt_classes
    pad = ((0, 0), (0, _NP - _N))
    anchors_t = jnp.pad(anchors.T, pad)
    deltas_t = jnp.pad(pred_deltas.T, pad)
    cls2 = jnp.pad(cls_preds.reshape(1, _N), pad)
    stride2 = jnp.pad(expanded_strides.reshape(1, _N), pad)

    costm, iouc, cdig, idig, maxiou = pl.pallas_call(
        _a1_body,
        out_shape=[jax.ShapeDtypeStruct((_G, _NP), jnp.float32),
                   jax.ShapeDtypeStruct((_G, _NP), jnp.float32),
                   jax.ShapeDtypeStruct((_G, _CH), jnp.float32),
                   jax.ShapeDtypeStruct((_G, _CH), jnp.float32),
                   jax.ShapeDtypeStruct((1, _NP), jnp.float32)],
    )(anchors_t, deltas_t, gt_boxes, cls2, stride2)

    trow_f = _make_sc_topk()(costm, iouc, cdig, idig)
    trow = jnp.reshape(trow_f, (_G, 16))

    out = pl.pallas_call(
        _c_body,
        out_shape=jax.ShapeDtypeStruct((8, _NP), jnp.float32),
    )(costm, iouc, trow, maxiou, gt_boxes)
    return out[:6, :_N].T
